# Initial kernel scaffold; baseline (speedup 1.0000x reference)
import jax
import jax.numpy as jnp
from jax.experimental import pallas as pl


def kernel(x, edge_index, W1, b1, W2, b2):
    raise NotImplementedError("write your pallas kernel here")


# speedup vs baseline: 10.2335x; 10.2335x over previous
import functools

import jax
import jax.numpy as jnp
from jax import lax
from jax.experimental import pallas as pl
from jax.experimental.pallas import tpu as pltpu
from jax.experimental.pallas import tpu_sc as plsc

N = 10000
D = 128
E = 320000

NC = 2
NS = 16
NW = NC * NS
C = 128
CHUNKS = 79
EPW = CHUNKS * C
E_PAD = NW * EPW

N_PAD = 10112
RPT = N_PAD // NS

_mesh = plsc.VectorSubcoreMesh(
    core_axis_name="c", subcore_axis_name="s", num_cores=NC, num_subcores=NS
)


@functools.partial(
    pl.kernel,
    out_type=jax.ShapeDtypeStruct((NC * N_PAD,), jnp.float32),
    mesh=_mesh,
    scratch_types=[
        pltpu.VMEM((C,), jnp.int32),
        pltpu.VMEM((C,), jnp.float32),
        pltpu.VMEM((RPT,), jnp.float32),
        pltpu.VMEM_SHARED((N_PAD,), jnp.float32),
    ],
)
def _sc_degree(dst_hbm, ones_hbm, zeros_hbm, out, dst_v, ones_v, stage_v, acc_sh):
    cid = lax.axis_index("c")
    sid = lax.axis_index("s")
    wid = sid * NC + cid
    r0 = sid * RPT
    pltpu.sync_copy(zeros_hbm.at[pl.ds(r0, RPT)], stage_v)
    pltpu.sync_copy(stage_v, acc_sh.at[pl.ds(r0, RPT)])
    pltpu.sync_copy(ones_hbm, ones_v)
    plsc.subcore_barrier()
    base0 = wid * EPW

    def chunk(j, carry):
        base = base0 + j * C
        pltpu.sync_copy(dst_hbm.at[pl.ds(base, C)], dst_v)
        pltpu.sync_copy(ones_v, acc_sh.at[dst_v], add=True)
        return carry

    lax.fori_loop(0, CHUNKS, chunk, 0)
    plsc.subcore_barrier()
    pltpu.sync_copy(acc_sh.at[pl.ds(r0, RPT)], stage_v)
    pltpu.sync_copy(stage_v, out.at[pl.ds(cid * N_PAD + r0, RPT)])


@functools.partial(
    pl.kernel,
    out_type=jax.ShapeDtypeStruct((NC, N_PAD, D), jnp.float32),
    mesh=_mesh,
    scratch_types=[
        pltpu.VMEM((C,), jnp.int32),
        pltpu.VMEM((C,), jnp.int32),
        pltpu.VMEM((C, D), jnp.float32),
        pltpu.VMEM_SHARED((N_PAD, D), jnp.float32),
        pltpu.SemaphoreType.DMA,
    ],
)
def _sc_aggregate(h_hbm, src_hbm, dst_hbm, zeros_hbm, out,
                  src_v, dst_v, rows_v, acc_sh, sem):
    cid = lax.axis_index("c")
    sid = lax.axis_index("s")
    wid = sid * NC + cid
    r0 = sid * RPT
    pltpu.sync_copy(zeros_hbm.at[pl.ds(r0, RPT)], acc_sh.at[pl.ds(r0, RPT)])
    plsc.subcore_barrier()
    base0 = wid * EPW

    def chunk(j, carry):
        base = base0 + j * C
        pltpu.sync_copy(src_hbm.at[pl.ds(base, C)], src_v)
        pltpu.sync_copy(dst_hbm.at[pl.ds(base, C)], dst_v)
        pltpu.async_copy(h_hbm.at[src_v], rows_v, sem).wait()
        pltpu.sync_copy(rows_v, acc_sh.at[dst_v], add=True)
        return carry

    lax.fori_loop(0, CHUNKS, chunk, 0)
    plsc.subcore_barrier()
    pltpu.sync_copy(acc_sh.at[pl.ds(r0, RPT)], out.at[cid, pl.ds(r0, RPT)])


R = 1000


def _tc_first_body(x_ref, w_ref, d0_ref, d1_ref, h_ref, dis_ref):
    deg = d0_ref[...] + d1_ref[...] + 1.0
    dis = lax.rsqrt(deg)
    h = jnp.dot(x_ref[...], w_ref[...], preferred_element_type=jnp.float32)
    h_ref[...] = h * dis
    dis_ref[...] = dis


def _tc_mid_body(a0_ref, a1_ref, h1_ref, dis_ref, b1_ref, w2_ref, h2_ref):
    dis = dis_ref[...]
    z = (a0_ref[...] + a1_ref[...] + h1_ref[...]) * dis + b1_ref[...]
    z = jnp.maximum(z, 0.0)
    h2_ref[...] = jnp.dot(z, w2_ref[...], preferred_element_type=jnp.float32) * dis


def _tc_last_body(c0_ref, c1_ref, h2_ref, dis_ref, b2_ref, out_ref):
    out_ref[...] = (c0_ref[...] + c1_ref[...] + h2_ref[...]) * dis_ref[...] \
        + b2_ref[...]


_row_blk = pl.BlockSpec((R, D), lambda i: (i, 0))
_dis_blk = pl.BlockSpec((R, 1), lambda i: (i, 0))
_mat_blk = pl.BlockSpec((D, D), lambda i: (0, 0))
_bias_blk = pl.BlockSpec((1, D), lambda i: (0, 0))

_tc_first = pl.pallas_call(
    _tc_first_body,
    grid=(N // R,),
    in_specs=[_row_blk, _mat_blk, _dis_blk, _dis_blk],
    out_specs=(_row_blk, _dis_blk),
    out_shape=(
        jax.ShapeDtypeStruct((N, D), jnp.float32),
        jax.ShapeDtypeStruct((N, 1), jnp.float32),
    ),
)

_tc_mid = pl.pallas_call(
    _tc_mid_body,
    grid=(N // R,),
    in_specs=[_row_blk, _row_blk, _row_blk, _dis_blk, _bias_blk, _mat_blk],
    out_specs=_row_blk,
    out_shape=jax.ShapeDtypeStruct((N, D), jnp.float32),
)

_tc_last = pl.pallas_call(
    _tc_last_body,
    grid=(N // R,),
    in_specs=[_row_blk, _row_blk, _row_blk, _dis_blk, _bias_blk],
    out_specs=_row_blk,
    out_shape=jax.ShapeDtypeStruct((N, D), jnp.float32),
)


def kernel(x, edge_index, W1, b1, W2, b2):
    src = edge_index[0].astype(jnp.int32)
    dst = edge_index[1].astype(jnp.int32)
    pad = E_PAD - E
    src_p = jnp.concatenate([src, jnp.zeros((pad,), jnp.int32)])
    dst_p = jnp.concatenate([dst, jnp.full((pad,), N, jnp.int32)])

    ones1 = jnp.ones((C,), jnp.float32)
    zeros1 = jnp.zeros((N_PAD,), jnp.float32)
    zerosD = jnp.zeros((N_PAD, D), jnp.float32)

    deg = _sc_degree(dst_p, ones1, zeros1).reshape(NC, N_PAD)
    h1, dis = _tc_first(x, W1, deg[0, :N].reshape(N, 1),
                        deg[1, :N].reshape(N, 1))
    a = _sc_aggregate(h1, src_p, dst_p, zerosD)
    h2 = _tc_mid(a[0, :N], a[1, :N], h1, dis, b1.reshape(1, D), W2)
    c = _sc_aggregate(h2, src_p, dst_p, zerosD)
    return _tc_last(c[0, :N], c[1, :N], h2, dis, b2.reshape(1, D))
